# baseline (speedup 1.0000x reference)
import functools
import math

import jax
import jax.numpy as jnp
from jax.experimental import pallas as pl
from jax.experimental.pallas import tpu as pltpu


def _dot(a, b, dims):
    return jax.lax.dot_general(a, b, dims, preferred_element_type=jnp.float32)


_LOG2E = 1.4426950408889634


def _fused_body(x_ref, h_ref, w_ref, b_ref, wn_ref, wnb_ref, cw_ref, cb_ref,
                o_ref, xbb_s, gef_s, xpb_s, xq_s, sqn2_s, dv_s, bmax_s,
                *, scale, qn):
    q = pl.program_id(1)
    c = scale * _LOG2E

    @pl.when(q == 0)
    def _batch_precompute():
        xb = x_ref[0]
        xbb = xb.astype(jnp.bfloat16)
        xbb_s[...] = xbb
        hf32 = h_ref[...].astype(jnp.float32)
        hf = hf32.astype(jnp.bfloat16)

        de = jnp.maximum(jnp.sum(hf32, axis=0), 1.0)
        dv_s[...] = jnp.maximum(jnp.sum(hf32, axis=1, keepdims=True), 1.0)

        edge_feat = _dot(hf, xbb, (((0,), (0,)), ((), ())))
        edge_feat = edge_feat / de[:, None]
        logit = jnp.sum(edge_feat * cw_ref[...], axis=1, keepdims=True)
        gamma = jax.nn.sigmoid(logit + cb_ref[0, 0])
        gef_s[...] = (gamma * edge_feat).astype(jnp.bfloat16)

        wnb16 = wn_ref[...].astype(jnp.bfloat16)
        xp = _dot(xbb, wnb16, (((1,), (0,)), ((), ()))) + wnb_ref[...]
        sqn2 = jnp.sum(xp * xp, axis=1, keepdims=True)
        sqn2_s[...] = sqn2
        bmax_s[...] = jnp.max(sqn2, axis=(0, 1), keepdims=True)
        xpb_s[...] = xp.astype(jnp.bfloat16)
        xq_s[...] = (xp * c).astype(jnp.bfloat16)

    qsl = pl.ds(q * qn, qn)

    hq = h_ref[qsl, :].astype(jnp.bfloat16)
    x_hyper = _dot(hq, gef_s[...], (((1,), (0,)), ((), ())))
    x_hyper = x_hyper / dv_s[qsl]

    s2 = _dot(xq_s[qsl], xpb_s[...], (((1,), (1,)), ((), ())))
    bound2 = jnp.sqrt(sqn2_s[qsl] * bmax_s[...]) * c
    eb = jnp.exp2(s2 - bound2).astype(jnp.bfloat16)
    den = jnp.sum(eb.astype(jnp.float32), axis=1, keepdims=True)
    num = _dot(eb, xbb_s[...], (((1,), (0,)), ((), ())))
    x_node = num / den

    hsum = (x_hyper + x_node).astype(jnp.bfloat16)
    w16 = w_ref[...].astype(jnp.bfloat16)
    out = _dot(hsum, w16, (((1,), (0,)), ((), ()))) + b_ref[...]
    o_ref[0] = out


def kernel(x, H, weight, bias, Wn_w, Wn_b, comp_w, comp_b, he_bias):
    B, N, F = x.shape
    O = weight.shape[1]
    E = H.shape[1]
    scale = 1.0 / math.sqrt(F)
    NQ = 2
    QN = N // NQ

    bias2 = bias.reshape(1, O)
    wnb2 = Wn_b.reshape(1, F)
    cw2 = comp_w.reshape(1, F)
    cb2 = (comp_b + he_bias).reshape(1, 1)

    grid = (B, NQ)
    out = pl.pallas_call(
        functools.partial(_fused_body, scale=scale, qn=QN),
        grid=grid,
        in_specs=[
            pl.BlockSpec((1, N, F), lambda b, q: (b, 0, 0)),
            pl.BlockSpec((N, E), lambda b, q: (0, 0)),
            pl.BlockSpec((F, O), lambda b, q: (0, 0)),
            pl.BlockSpec((1, O), lambda b, q: (0, 0)),
            pl.BlockSpec((F, F), lambda b, q: (0, 0)),
            pl.BlockSpec((1, F), lambda b, q: (0, 0)),
            pl.BlockSpec((1, F), lambda b, q: (0, 0)),
            pl.BlockSpec((1, 1), lambda b, q: (0, 0)),
        ],
        out_specs=pl.BlockSpec((1, QN, O), lambda b, q: (b, q, 0)),
        out_shape=jax.ShapeDtypeStruct((B, N, O), jnp.float32),
        scratch_shapes=[
            pltpu.VMEM((N, F), jnp.bfloat16),
            pltpu.VMEM((E, F), jnp.bfloat16),
            pltpu.VMEM((N, F), jnp.bfloat16),
            pltpu.VMEM((N, F), jnp.bfloat16),
            pltpu.VMEM((N, 1), jnp.float32),
            pltpu.VMEM((N, 1), jnp.float32),
            pltpu.VMEM((1, 1), jnp.float32),
        ],
        compiler_params=pltpu.CompilerParams(
            dimension_semantics=("arbitrary", "arbitrary"),
            vmem_limit_bytes=128 * 1024 * 1024,
        ),
    )(x, H, weight, bias2, Wn_w, wnb2, cw2, cb2)
    return out
